# Initial kernel scaffold; baseline (speedup 1.0000x reference)
import jax
import jax.numpy as jnp
from jax.experimental import pallas as pl


def kernel(x, Wr, W1, b1, W2, b2):
    raise NotImplementedError("write your pallas kernel here")


# speedup vs baseline: 3.3893x; 3.3893x over previous
import functools

import jax
import jax.numpy as jnp
from jax.experimental import pallas as pl
from jax.experimental.pallas import tpu as pltpu

H, E, FF, TOP_K = 1024, 8, 2048, 2
N = 2048
BT = 512
NB = N // BT


def _moe_kernel(x_ref, wr_ref, w1_ref, b1_ref, w2_ref, b2_ref,
                out_ref, aux_ref,
                i1_s, i2_s, wa_s, wb_s, cnt_s):
    e = pl.program_id(0)
    nb = pl.program_id(1)
    rows = pl.ds(nb * BT, BT)
    xb = x_ref[...]

    @pl.when(e == 0)
    def _router():
        lg = jnp.dot(xb, wr_ref[...], preferred_element_type=jnp.float32)
        ids = jax.lax.broadcasted_iota(jnp.int32, (BT, E), 1)
        m1 = jnp.max(lg, axis=1, keepdims=True)
        i1 = jnp.min(jnp.where(lg == m1, ids, E), axis=1, keepdims=True)
        masked = jnp.where(ids == i1, -jnp.inf, lg)
        m2 = jnp.max(masked, axis=1, keepdims=True)
        i2 = jnp.min(jnp.where(masked == m2, ids, E), axis=1, keepdims=True)
        r = jnp.exp(m2 - m1)
        wa = 1.0 / (1.0 + r)
        wb = 1.0 - wa
        i1_s[rows, :] = i1
        i2_s[rows, :] = i2
        wa_s[rows, :] = wa
        wb_s[rows, :] = wb
        cblk = (jnp.sum((ids == i1).astype(jnp.float32), axis=0, keepdims=True)
                + jnp.sum((ids == i2).astype(jnp.float32), axis=0, keepdims=True))

        @pl.when(nb == 0)
        def _():
            cnt_s[...] = cblk

        @pl.when(nb > 0)
        def _():
            cnt_s[...] = cnt_s[...] + cblk

    h = jnp.dot(xb, w1_ref[0], preferred_element_type=jnp.float32)
    h = h + b1_ref[0]
    h = 0.5 * h * (1.0 + jax.lax.erf(h * 0.7071067811865476))
    y = jnp.dot(h, w2_ref[0], preferred_element_type=jnp.float32) + b2_ref[0]

    w_col = (jnp.where(i1_s[rows, :] == e, wa_s[rows, :], 0.0)
             + jnp.where(i2_s[rows, :] == e, wb_s[rows, :], 0.0))
    contrib = y * w_col

    @pl.when(e == 0)
    def _():
        out_ref[rows, :] = contrib

    @pl.when(e > 0)
    def _():
        out_ref[rows, :] = out_ref[rows, :] + contrib

    @pl.when((e == E - 1) & (nb == NB - 1))
    def _aux():
        counts = cnt_s[...]
        load = counts / jnp.sum(counts)
        aux = 0.01 * jnp.sum(load * jnp.log(load + 1e-9), axis=1, keepdims=True)
        aux_ref[...] = aux


@jax.jit
def kernel(x, Wr, W1, b1, W2, b2):
    B, L, Hd = x.shape
    xf = x.reshape(-1, Hd)
    out, aux = pl.pallas_call(
        _moe_kernel,
        grid=(E, NB),
        in_specs=[
            pl.BlockSpec((BT, H), lambda e, nb: (nb, 0)),
            pl.BlockSpec((H, E), lambda e, nb: (0, 0)),
            pl.BlockSpec((1, H, FF), lambda e, nb: (e, 0, 0)),
            pl.BlockSpec((1, 1, FF), lambda e, nb: (e, 0, 0)),
            pl.BlockSpec((1, FF, H), lambda e, nb: (e, 0, 0)),
            pl.BlockSpec((1, 1, H), lambda e, nb: (e, 0, 0)),
        ],
        out_specs=[
            pl.BlockSpec((N, H), lambda e, nb: (0, 0)),
            pl.BlockSpec((1, 1), lambda e, nb: (0, 0)),
        ],
        out_shape=[
            jax.ShapeDtypeStruct((N, H), jnp.float32),
            jax.ShapeDtypeStruct((1, 1), jnp.float32),
        ],
        scratch_shapes=[
            pltpu.VMEM((N, 1), jnp.int32),
            pltpu.VMEM((N, 1), jnp.int32),
            pltpu.VMEM((N, 1), jnp.float32),
            pltpu.VMEM((N, 1), jnp.float32),
            pltpu.VMEM((1, E), jnp.float32),
        ],
    )(xf, Wr, W1, b1.reshape(E, 1, FF), W2, b2.reshape(E, 1, H))
    return out.reshape(B, L, Hd), aux.reshape(())
